# baseline (speedup 1.0000x reference)
import functools

import jax
import jax.numpy as jnp
from jax import lax
from jax.experimental import pallas as pl
from jax.experimental.pallas import tpu as pltpu
from jax.experimental.pallas import tpu_sc as plsc

F = 26
V = 100000
D = 16
B = 4096
NC = 2
NS = 16
NW = NC * NS
SPW = B // NW
W = 432
G4 = 4
NG = SPW // G4


def _take16(vec, idx):
    return vec.at[idx].get(mode="promise_in_bounds")


@functools.partial(
    pl.kernel,
    out_type=jax.ShapeDtypeStruct((B,), jnp.float32),
    mesh=plsc.VectorSubcoreMesh(core_axis_name="c", subcore_axis_name="s"),
    compiler_params=pltpu.CompilerParams(use_tc_tiling_on_sc=False),
    scratch_types=[
        pltpu.VMEM((SPW * F,), jnp.int32),
        pltpu.VMEM((G4 * F, W), jnp.float32),
        pltpu.VMEM((G4 * F, W), jnp.float32),
        pltpu.VMEM((SPW,), jnp.float32),
        pltpu.VMEM((16,), jnp.float32),
        pltpu.SemaphoreType.DMA,
        pltpu.SemaphoreType.DMA,
    ],
)
def _ffm_sc(x_hbm, tab_hbm, bias_hbm, out_hbm,
            x_v, slabs0_v, slabs1_v, out_v, bias_v, sem0, sem1):
    wid = lax.axis_index("s") * NC + lax.axis_index("c")
    base = wid * SPW

    pltpu.sync_copy(bias_hbm, bias_v)
    pltpu.sync_copy(x_hbm.at[pl.ds(base * F, SPW * F)], x_v)

    lane = lax.iota(jnp.int32, 16)

    def fire(g, slabs_ref, sem):
        idx = x_v.at[pl.ds(g * (G4 * F), G4 * F)]
        pltpu.async_copy(tab_hbm.at[idx], slabs_ref, sem)

    def wait(g, slabs_ref, sem):
        idx = x_v.at[pl.ds(g * (G4 * F), G4 * F)]
        pltpu.make_async_copy(tab_hbm.at[idx], slabs_ref, sem).wait()

    def compute(g, slabs_ref, out_vec):
        for r in range(G4):
            b = g * G4 + r
            acc = jnp.zeros((16,), jnp.float32)
            for i in range(F - 1):
                for j in range(i + 1, F):
                    acc = acc + (slabs_ref[r * F + i, pl.ds(j * D, D)] *
                                 slabs_ref[r * F + j, pl.ds(i * D, D)])
            accf = slabs_ref[r * F, pl.ds(F * D, D)]
            for s in range(1, F):
                accf = accf + slabs_ref[r * F + s, pl.ds(F * D, D)]
            acc = acc + jnp.where(lane == 0, accf, 0.0)
            for sh in (8, 4, 2, 1):
                acc = acc + _take16(acc, lane ^ sh)
            out_vec = jnp.where(lane == b % 16, acc, out_vec)
            out_v[pl.ds((b // 16) * 16, 16)] = out_vec
        return out_vec

    fire(0, slabs0_v, sem0)

    def pair_body(gg, out_vec):
        g0 = 2 * gg
        g1 = g0 + 1
        fire(g1, slabs1_v, sem1)
        wait(g0, slabs0_v, sem0)
        out_vec = compute(g0, slabs0_v, out_vec)
        fire(jnp.minimum(g1 + 1, NG - 1), slabs0_v, sem0)
        wait(g1, slabs1_v, sem1)
        return compute(g1, slabs1_v, out_vec)

    lax.fori_loop(0, NG // 2, pair_body, jnp.zeros((16,), jnp.float32))
    wait(NG - 1, slabs0_v, sem0)

    bb = bias_v[:]
    for g in range(SPW // 16):
        zz = out_v[pl.ds(g * 16, 16)] + bb
        out_v[pl.ds(g * 16, 16)] = 1.0 / (1.0 + jnp.exp(-zz))
    pltpu.sync_copy(out_v, out_hbm.at[pl.ds(base, SPW)])


def kernel(x, emb_tables, fc_table, bias):
    xflat = x.astype(jnp.int32).reshape(B * F)
    tab_aug = jnp.concatenate(
        [emb_tables.transpose(1, 0, 2).reshape(V, F * D),
         fc_table.astype(jnp.float32),
         jnp.zeros((V, W - F * D - 1), jnp.float32)], axis=1)
    bias16 = jnp.broadcast_to(bias.astype(jnp.float32), (16,))
    return _ffm_sc(xflat, tab_aug, bias16)
